# baseline (speedup 1.0000x reference)
import functools

import jax
import jax.numpy as jnp
from jax import lax
from jax.experimental import pallas as pl
from jax.experimental.pallas import tpu as pltpu
from jax.experimental.pallas import tpu_sc as plsc

_B = 16384
_E = 64
_MLP_BLK = 2048


def _sc_gather_body(nc, bpw, uidx_hbm, midx_hbm, eu_hbm, em_hbm,
                    u_out, m_out, idx_u, idx_m, rows, sem):
    wid = lax.axis_index("s") * nc + lax.axis_index("c")
    base = wid * bpw
    pltpu.sync_copy(uidx_hbm.at[pl.ds(base, bpw)], idx_u)
    pltpu.sync_copy(midx_hbm.at[pl.ds(base, bpw)], idx_m)
    pltpu.async_copy(eu_hbm.at[idx_u], rows, sem).wait()
    pltpu.sync_copy(rows, u_out.at[pl.ds(base, bpw)])
    pltpu.async_copy(em_hbm.at[idx_m], rows, sem).wait()
    pltpu.sync_copy(rows, m_out.at[pl.ds(base, bpw)])


def _sc_gather(uidx, midx, eu_lines, em_lines):
    info = plsc.get_sparse_core_info()
    nc, ns = info.num_cores, info.num_subcores
    nw = nc * ns
    bpw = _B // nw
    mesh = plsc.VectorSubcoreMesh(core_axis_name="c", subcore_axis_name="s")
    k = pl.kernel(
        functools.partial(_sc_gather_body, nc, bpw),
        out_type=(jax.ShapeDtypeStruct((_B, 2 * _E), jnp.float32),
                  jax.ShapeDtypeStruct((_B, 2 * _E), jnp.float32)),
        mesh=mesh,
        scratch_types=[
            pltpu.VMEM((bpw,), jnp.int32),
            pltpu.VMEM((bpw,), jnp.int32),
            pltpu.VMEM((bpw, 2 * _E), jnp.float32),
            pltpu.SemaphoreType.DMA,
        ],
    )
    return k(uidx, midx, eu_lines, em_lines)


def _mlp_body(lu_ref, lm_ref, pu_ref, pm_ref, w1u_ref, w1m_ref, b1_ref,
              w2_ref, b2_ref, w3_ref, b3_ref, out_ref):
    lu = lu_ref[...]
    lm = lm_ref[...]
    pu = pu_ref[...]
    pm = pm_ref[...]
    ue = lu[:, :_E] + pu * (lu[:, _E:] - lu[:, :_E])
    me = lm[:, :_E] + pm * (lm[:, _E:] - lm[:, :_E])
    h = jnp.dot(ue, w1u_ref[...], preferred_element_type=jnp.float32)
    h = h + jnp.dot(me, w1m_ref[...], preferred_element_type=jnp.float32)
    h = jnp.maximum(h + b1_ref[...], 0.0)
    h = jnp.maximum(
        jnp.dot(h, w2_ref[...], preferred_element_type=jnp.float32) + b2_ref[...],
        0.0)
    o = jnp.dot(h, w3_ref[...], preferred_element_type=jnp.float32) + b3_ref[...]
    out_ref[...] = jnp.maximum(o[:, 0], 0.0)


def _mlp(u_lines, m_lines, pu, pm, W1, b1, W2, b2, W3, b3):
    w1u, w1m = W1[:_E], W1[_E:]
    grid = _B // _MLP_BLK
    line_spec = pl.BlockSpec((_MLP_BLK, 2 * _E), lambda i: (i, 0))
    par_spec = pl.BlockSpec((_MLP_BLK, 1), lambda i: (i, 0))

    def full(shape):
        return pl.BlockSpec(shape, lambda i: (0, 0))

    return pl.pallas_call(
        _mlp_body,
        grid=(grid,),
        in_specs=[line_spec, line_spec, par_spec, par_spec,
                  full((_E, 64)), full((_E, 64)), full((1, 64)),
                  full((64, 16)), full((1, 16)), full((16, 1)), full((1, 1))],
        out_specs=pl.BlockSpec((_MLP_BLK,), lambda i: (i,)),
        out_shape=jax.ShapeDtypeStruct((_B,), jnp.float32),
    )(u_lines, m_lines, pu, pm, w1u, w1m, b1.reshape(1, -1), W2,
      b2.reshape(1, -1), W3, b3.reshape(1, -1))


def kernel(users, movies, emb_users, emb_movies, W1, b1, W2, b2, W3, b3):
    users = users.astype(jnp.int32)
    movies = movies.astype(jnp.int32)
    eu_lines = emb_users.reshape(-1, 2 * _E)
    em_lines = emb_movies.reshape(-1, 2 * _E)
    pu = (users & 1).astype(jnp.float32).reshape(_B, 1)
    pm = (movies & 1).astype(jnp.float32).reshape(_B, 1)
    u_lines, m_lines = _sc_gather(users >> 1, movies >> 1, eu_lines, em_lines)
    return _mlp(u_lines, m_lines, pu, pm, W1, b1, W2, b2, W3, b3)
